# Initial kernel scaffold; baseline (speedup 1.0000x reference)
import jax
import jax.numpy as jnp
from jax.experimental import pallas as pl


def kernel(x, pos, Ws1, Ws2, Wl1, Wl2, bs1, bs2, bl1, bl2, gs1, gs2, gl1, gl2, bes1, bes2, bel1, bel2):
    raise NotImplementedError("write your pallas kernel here")


# speedup vs baseline: 15.2020x; 15.2020x over previous
import functools

import jax
import jax.numpy as jnp
from jax import lax
from jax.experimental import pallas as pl
from jax.experimental.pallas import tpu as pltpu
from jax.experimental.pallas import tpu_sc as plsc

_EPS = 1e-5
_N = 4096
_C = 128
_R = 512
_T = 256
_BIGI = 2 ** 30
_HI = lax.Precision.HIGHEST


def _proj_body(xT_ref, w1_ref, b1_ref, aT_ref, bT_ref):
    xT = xT_ref[...]
    w1 = w1_ref[...]
    wa = w1[:, :_C]
    wb = w1[:, _C:]
    dn = (((1,), (1,)), ((), ()))
    aT_ref[...] = (lax.dot_general(xT, wa - wb, dn, precision=_HI)
                   + b1_ref[...])
    bT_ref[...] = lax.dot_general(xT, wb, dn, precision=_HI)


def _proj(xT, W1, b1):
    return pl.pallas_call(
        _proj_body,
        out_shape=(jax.ShapeDtypeStruct((_N, _C), jnp.float32),
                   jax.ShapeDtypeStruct((_N, _C), jnp.float32)),
    )(xT, W1, b1.reshape(1, _C))


def _knn_body(pT_ref, p_ref, idx_ref):
    i = pl.program_id(0)
    pr = pT_ref[...]
    pc = p_ref[...]
    sqr = jnp.sum(pr * pr, axis=1, keepdims=True)
    sqc = jnp.sum(pc * pc, axis=0, keepdims=True)
    dot = lax.dot_general(pr.astype(jnp.bfloat16), pc.astype(jnp.bfloat16),
                          (((1,), (0,)), ((), ())),
                          preferred_element_type=jnp.float32)
    d = jnp.sqrt(jnp.maximum(sqr + sqc - 2.0 * dot, 0.0))
    ci = lax.broadcasted_iota(jnp.int32, (_R, _N), 1)
    li = lax.broadcasted_iota(jnp.int32, (_R, 128), 1)
    ri = lax.broadcasted_iota(jnp.int32, (_R, 1), 0)
    idxacc = jnp.zeros((_R, 128), jnp.int32)
    for kk in range(12):
        m = jnp.min(d, axis=1, keepdims=True)
        am = jnp.min(jnp.where(d == m, ci, _BIGI), axis=1, keepdims=True)
        store = (i * _R + ri) if kk == 0 else am
        idxacc = jnp.where(li == kk, jnp.broadcast_to(store, (_R, 128)),
                           idxacc)
        d = jnp.where(ci == am, jnp.float32(jnp.inf), d)
    idx_ref[...] = idxacc


def _knn(pT, p):
    return pl.pallas_call(
        _knn_body,
        grid=(_N // _R,),
        in_specs=[pl.BlockSpec((_R, 3), lambda i: (i, 0)),
                  pl.BlockSpec((3, _N), lambda i: (0, 0))],
        out_specs=pl.BlockSpec((_R, 128), lambda i: (i, 0)),
        out_shape=jax.ShapeDtypeStruct((_N, 128), jnp.int32),
    )(pT, p)


_NW = 32
_CHK = 768


def _sc_gather_body(tabs_ref, tabl_ref, idx6_ref, idx12_ref, gs_ref, gl_ref,
                    idx_v, rows_v, sem):
    wid = lax.axis_index("s") * 2 + lax.axis_index("c")
    base = wid * _CHK
    pltpu.sync_copy(idx6_ref.at[pl.ds(base, _CHK)], idx_v)
    pltpu.async_copy(tabs_ref.at[idx_v], rows_v, sem).wait()
    pltpu.sync_copy(rows_v, gs_ref.at[pl.ds(base, _CHK)])
    for c in range(2):
        base = wid * 2 * _CHK + c * _CHK
        pltpu.sync_copy(idx12_ref.at[pl.ds(base, _CHK)], idx_v)
        pltpu.async_copy(tabl_ref.at[idx_v], rows_v, sem).wait()
        pltpu.sync_copy(rows_v, gl_ref.at[pl.ds(base, _CHK)])


def _sc_gather(bsT, blT, flat6, flat12):
    mesh = plsc.VectorSubcoreMesh(core_axis_name="c", subcore_axis_name="s")
    f = pl.kernel(
        _sc_gather_body, mesh=mesh,
        out_type=(jax.ShapeDtypeStruct((_N * 6, _C), jnp.float32),
                  jax.ShapeDtypeStruct((_N * 12, _C), jnp.float32)),
        scratch_types=[pltpu.VMEM((_CHK,), jnp.int32),
                       pltpu.VMEM((_CHK, _C), jnp.float32),
                       pltpu.SemaphoreType.DMA],
    )
    return f(bsT, blT, flat6, flat12)


def _stats_body(k, g_ref, aT_ref, out_ref, acc_ref):
    i = pl.program_id(0)

    @pl.when(i == 0)
    def _():
        acc_ref[...] = jnp.zeros_like(acc_ref)

    g = g_ref[...]
    a = aT_ref[...]
    h = (g.reshape(_T, k, _C) + a[:, None, :]).reshape(_T * k, _C)
    acc_ref[0:1, :] += jnp.sum(h, axis=0, keepdims=True)
    acc_ref[1:2, :] += jnp.sum(h * h, axis=0, keepdims=True)

    @pl.when(i == pl.num_programs(0) - 1)
    def _():
        out_ref[...] = acc_ref[...]


def _stats(k, g, aT):
    return pl.pallas_call(
        functools.partial(_stats_body, k),
        grid=(_N // _T,),
        in_specs=[pl.BlockSpec((_T * k, _C), lambda i: (i, 0)),
                  pl.BlockSpec((_T, _C), lambda i: (i, 0))],
        out_specs=pl.BlockSpec((2, _C), lambda i: (0, 0)),
        out_shape=jax.ShapeDtypeStruct((2, _C), jnp.float32),
        scratch_shapes=[pltpu.VMEM((2, _C), jnp.float32)],
    )(g, aT)


def _main_body(k, g_ref, aT_ref, s1_ref, w2_ref, vec_ref, m2_ref, s2_ref,
               acc_ref):
    i = pl.program_id(0)

    @pl.when(i == 0)
    def _():
        acc_ref[...] = jnp.zeros_like(acc_ref)

    nk = jnp.float32(_N * k)
    s1 = s1_ref[...]
    m1 = s1[0:1, :] / nk
    v1 = s1[1:2, :] / nk - m1 * m1
    g1 = vec_ref[0:1, :]
    be1 = vec_ref[1:2, :]
    b2 = vec_ref[2:3, :]
    sc1 = g1 * lax.rsqrt(v1 + _EPS)
    sh1 = be1 - m1 * sc1
    g = g_ref[...]
    a = aT_ref[...]
    h = (g.reshape(_T, k, _C) + a[:, None, :]).reshape(_T * k, _C)
    h = jnp.maximum(h * sc1 + sh1, 0.0)
    h2 = lax.dot_general(h, w2_ref[...], (((1,), (1,)), ((), ())),
                         precision=_HI) + b2
    acc_ref[0:1, :] += jnp.sum(h2, axis=0, keepdims=True)
    acc_ref[1:2, :] += jnp.sum(h2 * h2, axis=0, keepdims=True)
    m2_ref[...] = jnp.max(h2.reshape(_T, k, _C), axis=1)

    @pl.when(i == pl.num_programs(0) - 1)
    def _():
        s2_ref[...] = acc_ref[...]


def _main(k, g, aT, s1, W2, vec):
    return pl.pallas_call(
        functools.partial(_main_body, k),
        grid=(_N // _T,),
        in_specs=[pl.BlockSpec((_T * k, _C), lambda i: (i, 0)),
                  pl.BlockSpec((_T, _C), lambda i: (i, 0)),
                  pl.BlockSpec((2, _C), lambda i: (0, 0)),
                  pl.BlockSpec((_C, _C), lambda i: (0, 0)),
                  pl.BlockSpec((5, _C), lambda i: (0, 0))],
        out_specs=(pl.BlockSpec((_T, _C), lambda i: (i, 0)),
                   pl.BlockSpec((2, _C), lambda i: (0, 0))),
        out_shape=(jax.ShapeDtypeStruct((_N, _C), jnp.float32),
                   jax.ShapeDtypeStruct((2, _C), jnp.float32)),
        scratch_shapes=[pltpu.VMEM((2, _C), jnp.float32)],
    )(g, aT, s1, W2, vec)


def _final_body(k, m2_ref, s2_ref, vec_ref, out_ref):
    nk = jnp.float32(_N * k)
    s2 = s2_ref[...]
    m2 = s2[0:1, :] / nk
    v2 = s2[1:2, :] / nk - m2 * m2
    g2 = vec_ref[3:4, :]
    be2 = vec_ref[4:5, :]
    sc2 = g2 * lax.rsqrt(v2 + _EPS)
    sh2 = be2 - m2 * sc2
    o = jnp.maximum(m2_ref[...] * sc2 + sh2, 0.0)
    out_ref[...] = o.T


def _final(k, m2, s2, vec):
    return pl.pallas_call(
        functools.partial(_final_body, k),
        grid=(_N // _T,),
        in_specs=[pl.BlockSpec((_T, _C), lambda i: (i, 0)),
                  pl.BlockSpec((2, _C), lambda i: (0, 0)),
                  pl.BlockSpec((5, _C), lambda i: (0, 0))],
        out_specs=pl.BlockSpec((_C, _T), lambda i: (0, i)),
        out_shape=jax.ShapeDtypeStruct((_C, _N), jnp.float32),
    )(m2, s2, vec)


def _stream(k, g, aT, W2, vec):
    s1 = _stats(k, g, aT)
    m2, s2 = _main(k, g, aT, s1, W2, vec)
    return _final(k, m2, s2, vec)


def kernel(x, pos, Ws1, Ws2, Wl1, Wl2, bs1, bs2, bl1, bl2, gs1, gs2, gl1,
           gl2, bes1, bes2, bel1, bel2):
    xT = x[0].T
    p = pos[0].astype(jnp.float32)
    pT = p.T

    asT, bsT = _proj(xT, Ws1, bs1)
    alT, blT = _proj(xT, Wl1, bl1)

    idx = _knn(pT, p)
    flat6 = idx[:, :6].reshape(-1)
    flat12 = idx[:, :12].reshape(-1)
    gs, gl = _sc_gather(bsT, blT, flat6, flat12)

    vec_s = jnp.stack([gs1, bes1, bs2, gs2, bes2], axis=0)
    vec_l = jnp.stack([gl1, bel1, bl2, gl2, bel2], axis=0)
    outs = _stream(6, gs, asT, Ws2, vec_s)
    outl = _stream(12, gl, alT, Wl2, vec_l)
    return jnp.concatenate([outs, outl], axis=0)[None, :, :]


# speedup vs baseline: 16.6653x; 1.0963x over previous
import functools

import jax
import jax.numpy as jnp
from jax import lax
from jax.experimental import pallas as pl
from jax.experimental.pallas import tpu as pltpu
from jax.experimental.pallas import tpu_sc as plsc

_EPS = 1e-5
_N = 4096
_C = 128
_R = 512
_T = 256
_BIGI = 2 ** 30
_HI = lax.Precision.HIGHEST


def _proj_body(xT_ref, w1_ref, b1_ref, aT_ref, bT_ref):
    xT = xT_ref[...]
    w1 = w1_ref[...]
    wa = w1[:, :_C]
    wb = w1[:, _C:]
    dn = (((1,), (1,)), ((), ()))
    xb = xT.astype(jnp.bfloat16)
    aT_ref[...] = (lax.dot_general(xb, (wa - wb).astype(jnp.bfloat16), dn,
                                   preferred_element_type=jnp.float32)
                   + b1_ref[...])
    bT_ref[...] = lax.dot_general(xb, wb.astype(jnp.bfloat16), dn,
                                  preferred_element_type=jnp.float32)


def _proj(xT, W1, b1):
    return pl.pallas_call(
        _proj_body,
        out_shape=(jax.ShapeDtypeStruct((_N, _C), jnp.float32),
                   jax.ShapeDtypeStruct((_N, _C), jnp.float32)),
    )(xT, W1, b1.reshape(1, _C))


def _knn_body(pT_ref, p_ref, idx_ref):
    i = pl.program_id(0)
    pr = pT_ref[...]
    pc = p_ref[...]
    sqr = jnp.sum(pr * pr, axis=1, keepdims=True)
    sqc = jnp.sum(pc * pc, axis=0, keepdims=True)
    dot = lax.dot_general(pr.astype(jnp.bfloat16), pc.astype(jnp.bfloat16),
                          (((1,), (0,)), ((), ())),
                          preferred_element_type=jnp.float32)
    d = jnp.sqrt(jnp.maximum(sqr + sqc - 2.0 * dot, 0.0))
    IMAX = jnp.int32(2147483647)
    NL = 128
    NCH = _N // NL
    k1 = jnp.full((_R, NL), IMAX, jnp.int32)
    k2 = jnp.full((_R, NL), IMAX, jnp.int32)
    k3 = jnp.full((_R, NL), IMAX, jnp.int32)
    k4 = jnp.full((_R, NL), IMAX, jnp.int32)
    k5 = jnp.full((_R, NL), IMAX, jnp.int32)
    for c in range(NCH):
        bits = lax.bitcast_convert_type(d[:, c * NL:(c + 1) * NL], jnp.int32)
        x = (bits & jnp.int32(-32)) | jnp.int32(c)
        t = jnp.minimum(k1, x); x = jnp.maximum(k1, x); k1 = t
        t = jnp.minimum(k2, x); x = jnp.maximum(k2, x); k2 = t
        t = jnp.minimum(k3, x); x = jnp.maximum(k3, x); k3 = t
        t = jnp.minimum(k4, x); x = jnp.maximum(k4, x); k4 = t
        k5 = jnp.minimum(k5, x)
    li = lax.broadcasted_iota(jnp.int32, (_R, NL), 1)
    ri = lax.broadcasted_iota(jnp.int32, (_R, 1), 0)
    idxacc = jnp.zeros((_R, NL), jnp.int32)
    for kk in range(12):
        m = jnp.min(k1, axis=1, keepdims=True)
        lane = jnp.min(jnp.where(k1 == m, li, _BIGI), axis=1,
                       keepdims=True)
        col = ((m & 31) << 7) | lane
        store = (i * _R + ri) if kk == 0 else col
        idxacc = jnp.where(li == kk, jnp.broadcast_to(store, (_R, NL)),
                           idxacc)
        pop = li == lane
        k1 = jnp.where(pop, k2, k1)
        k2 = jnp.where(pop, k3, k2)
        k3 = jnp.where(pop, k4, k3)
        k4 = jnp.where(pop, k5, k4)
        k5 = jnp.where(pop, IMAX, k5)
    idx_ref[...] = idxacc


def _knn(pT, p):
    return pl.pallas_call(
        _knn_body,
        grid=(_N // _R,),
        in_specs=[pl.BlockSpec((_R, 3), lambda i: (i, 0)),
                  pl.BlockSpec((3, _N), lambda i: (0, 0))],
        out_specs=pl.BlockSpec((_R, 128), lambda i: (i, 0)),
        out_shape=jax.ShapeDtypeStruct((_N, 128), jnp.int32),
    )(pT, p)


_NW = 32
_CHK = 768


def _sc_gather_body(tabs_ref, tabl_ref, idx6_ref, idx12_ref, gs_ref, gl_ref,
                    idx_v, rows_v, sem):
    wid = lax.axis_index("s") * 2 + lax.axis_index("c")
    base = wid * _CHK
    pltpu.sync_copy(idx6_ref.at[pl.ds(base, _CHK)], idx_v)
    pltpu.async_copy(tabs_ref.at[idx_v], rows_v, sem).wait()
    pltpu.sync_copy(rows_v, gs_ref.at[pl.ds(base, _CHK)])
    for c in range(2):
        base = wid * 2 * _CHK + c * _CHK
        pltpu.sync_copy(idx12_ref.at[pl.ds(base, _CHK)], idx_v)
        pltpu.async_copy(tabl_ref.at[idx_v], rows_v, sem).wait()
        pltpu.sync_copy(rows_v, gl_ref.at[pl.ds(base, _CHK)])


def _sc_gather(bsT, blT, flat6, flat12):
    mesh = plsc.VectorSubcoreMesh(core_axis_name="c", subcore_axis_name="s")
    f = pl.kernel(
        _sc_gather_body, mesh=mesh,
        out_type=(jax.ShapeDtypeStruct((_N * 6, _C), jnp.float32),
                  jax.ShapeDtypeStruct((_N * 12, _C), jnp.float32)),
        scratch_types=[pltpu.VMEM((_CHK,), jnp.int32),
                       pltpu.VMEM((_CHK, _C), jnp.float32),
                       pltpu.SemaphoreType.DMA],
    )
    return f(bsT, blT, flat6, flat12)


def _stats_body(k, g_ref, aT_ref, out_ref, acc_ref):
    i = pl.program_id(0)

    @pl.when(i == 0)
    def _():
        acc_ref[...] = jnp.zeros_like(acc_ref)

    g = g_ref[...]
    a = aT_ref[...]
    h = (g.reshape(_T, k, _C) + a[:, None, :]).reshape(_T * k, _C)
    acc_ref[0:1, :] += jnp.sum(h, axis=0, keepdims=True)
    acc_ref[1:2, :] += jnp.sum(h * h, axis=0, keepdims=True)

    @pl.when(i == pl.num_programs(0) - 1)
    def _():
        out_ref[...] = acc_ref[...]


def _stats(k, g, aT):
    return pl.pallas_call(
        functools.partial(_stats_body, k),
        grid=(_N // _T,),
        in_specs=[pl.BlockSpec((_T * k, _C), lambda i: (i, 0)),
                  pl.BlockSpec((_T, _C), lambda i: (i, 0))],
        out_specs=pl.BlockSpec((2, _C), lambda i: (0, 0)),
        out_shape=jax.ShapeDtypeStruct((2, _C), jnp.float32),
        scratch_shapes=[pltpu.VMEM((2, _C), jnp.float32)],
    )(g, aT)


def _main_body(k, g_ref, aT_ref, s1_ref, w2_ref, vec_ref, m2_ref, s2_ref,
               acc_ref):
    i = pl.program_id(0)

    @pl.when(i == 0)
    def _():
        acc_ref[...] = jnp.zeros_like(acc_ref)

    nk = jnp.float32(_N * k)
    s1 = s1_ref[...]
    m1 = s1[0:1, :] / nk
    v1 = s1[1:2, :] / nk - m1 * m1
    g1 = vec_ref[0:1, :]
    be1 = vec_ref[1:2, :]
    b2 = vec_ref[2:3, :]
    sc1 = g1 * lax.rsqrt(v1 + _EPS)
    sh1 = be1 - m1 * sc1
    g = g_ref[...]
    a = aT_ref[...]
    h = (g.reshape(_T, k, _C) + a[:, None, :]).reshape(_T * k, _C)
    h = jnp.maximum(h * sc1 + sh1, 0.0)
    h2 = lax.dot_general(h.astype(jnp.bfloat16),
                         w2_ref[...].astype(jnp.bfloat16),
                         (((1,), (1,)), ((), ())),
                         preferred_element_type=jnp.float32) + b2
    acc_ref[0:1, :] += jnp.sum(h2, axis=0, keepdims=True)
    acc_ref[1:2, :] += jnp.sum(h2 * h2, axis=0, keepdims=True)
    m2_ref[...] = jnp.max(h2.reshape(_T, k, _C), axis=1)

    @pl.when(i == pl.num_programs(0) - 1)
    def _():
        s2_ref[...] = acc_ref[...]


def _main(k, g, aT, s1, W2, vec):
    return pl.pallas_call(
        functools.partial(_main_body, k),
        grid=(_N // _T,),
        in_specs=[pl.BlockSpec((_T * k, _C), lambda i: (i, 0)),
                  pl.BlockSpec((_T, _C), lambda i: (i, 0)),
                  pl.BlockSpec((2, _C), lambda i: (0, 0)),
                  pl.BlockSpec((_C, _C), lambda i: (0, 0)),
                  pl.BlockSpec((5, _C), lambda i: (0, 0))],
        out_specs=(pl.BlockSpec((_T, _C), lambda i: (i, 0)),
                   pl.BlockSpec((2, _C), lambda i: (0, 0))),
        out_shape=(jax.ShapeDtypeStruct((_N, _C), jnp.float32),
                   jax.ShapeDtypeStruct((2, _C), jnp.float32)),
        scratch_shapes=[pltpu.VMEM((2, _C), jnp.float32)],
    )(g, aT, s1, W2, vec)


def _final_body(k, m2_ref, s2_ref, vec_ref, out_ref):
    nk = jnp.float32(_N * k)
    s2 = s2_ref[...]
    m2 = s2[0:1, :] / nk
    v2 = s2[1:2, :] / nk - m2 * m2
    g2 = vec_ref[3:4, :]
    be2 = vec_ref[4:5, :]
    sc2 = g2 * lax.rsqrt(v2 + _EPS)
    sh2 = be2 - m2 * sc2
    o = jnp.maximum(m2_ref[...] * sc2 + sh2, 0.0)
    out_ref[...] = o.T


def _final(k, m2, s2, vec):
    return pl.pallas_call(
        functools.partial(_final_body, k),
        grid=(_N // _T,),
        in_specs=[pl.BlockSpec((_T, _C), lambda i: (i, 0)),
                  pl.BlockSpec((2, _C), lambda i: (0, 0)),
                  pl.BlockSpec((5, _C), lambda i: (0, 0))],
        out_specs=pl.BlockSpec((_C, _T), lambda i: (0, i)),
        out_shape=jax.ShapeDtypeStruct((_C, _N), jnp.float32),
    )(m2, s2, vec)


def _stream(k, g, aT, W2, vec):
    s1 = _stats(k, g, aT)
    m2, s2 = _main(k, g, aT, s1, W2, vec)
    return _final(k, m2, s2, vec)


def kernel(x, pos, Ws1, Ws2, Wl1, Wl2, bs1, bs2, bl1, bl2, gs1, gs2, gl1,
           gl2, bes1, bes2, bel1, bel2):
    xT = x[0].T
    p = pos[0].astype(jnp.float32)
    pT = p.T

    asT, bsT = _proj(xT, Ws1, bs1)
    alT, blT = _proj(xT, Wl1, bl1)

    idx = _knn(pT, p)
    flat6 = idx[:, :6].reshape(-1)
    flat12 = idx[:, :12].reshape(-1)
    gs, gl = _sc_gather(bsT, blT, flat6, flat12)

    vec_s = jnp.stack([gs1, bes1, bs2, gs2, bes2], axis=0)
    vec_l = jnp.stack([gl1, bel1, bl2, gl2, bel2], axis=0)
    outs = _stream(6, gs, asT, Ws2, vec_s)
    outl = _stream(12, gl, alT, Wl2, vec_l)
    return jnp.concatenate([outs, outl], axis=0)[None, :, :]
